# baseline (speedup 1.0000x reference)
import functools

import jax
import jax.numpy as jnp
from jax import lax
from jax.experimental import pallas as pl
from jax.experimental.pallas import tpu as pltpu
from jax.experimental.pallas import tpu_sc as plsc


def _make_sc_gather(V, D, B):
    info = plsc.get_sparse_core_info()
    NC, NS = info.num_cores, info.num_subcores
    NW = NC * NS
    assert B % NW == 0
    b_per_w = B // NW
    CH = 128 if b_per_w % 128 == 0 else b_per_w
    n_ch = b_per_w // CH
    mesh = plsc.VectorSubcoreMesh(core_axis_name="c", subcore_axis_name="s")

    @functools.partial(
        pl.kernel,
        mesh=mesh,
        out_type=jax.ShapeDtypeStruct((B, D), jnp.float32),
        scratch_types=[
            pltpu.VMEM((b_per_w,), jnp.int32),
            pltpu.VMEM((b_per_w, D), jnp.float32),
            pltpu.SemaphoreType.DMA,
        ],
    )
    def sc_gather(table_hbm, idx_hbm, out_hbm, idx_v, rows_v, sem):
        wid = lax.axis_index("s") * NC + lax.axis_index("c")
        base = wid * b_per_w
        pltpu.sync_copy(idx_hbm.at[pl.ds(base, b_per_w)], idx_v)
        copies = []
        for j in range(n_ch):
            copies.append(pltpu.async_copy(
                table_hbm.at[idx_v.at[pl.ds(j * CH, CH)]],
                rows_v.at[pl.ds(j * CH, CH)],
                sem,
            ))
        for c in copies:
            c.wait()
        pltpu.sync_copy(rows_v, out_hbm.at[pl.ds(base, b_per_w)])

    return sc_gather


def _mlp_body(x_ref, w1_ref, b1_ref, w2_ref, b2_ref, o_hbm, scr, sem):
    i = pl.program_id(0)
    ni = pl.num_programs(0)
    blk = x_ref.shape[0]
    nt, td = b2_ref.shape
    x = x_ref[...].astype(jnp.bfloat16)
    h = jnp.dot(x, w1_ref[...], preferred_element_type=jnp.float32) + b1_ref[...]
    h = (h * jax.nn.sigmoid(h)).astype(jnp.bfloat16)
    for t in range(nt):
        slot = t % 2
        gt = i * nt + t

        @pl.when(gt >= 2)
        def _():
            pltpu.make_async_copy(
                scr.at[slot], o_hbm.at[pl.ds(0, blk), 0, :], sem.at[slot]
            ).wait()

        o = jnp.dot(h, w2_ref[:, t * td:(t + 1) * td],
                    preferred_element_type=jnp.float32) + b2_ref[t, :]
        scr[slot] = o
        pltpu.make_async_copy(
            scr.at[slot], o_hbm.at[pl.ds(i * blk, blk), t, :], sem.at[slot]
        ).start()

    @pl.when(i == ni - 1)
    def _():
        for slot in range(2):
            pltpu.make_async_copy(
                scr.at[slot], o_hbm.at[pl.ds(0, blk), 0, :], sem.at[slot]
            ).wait()


def _tc_mlp(x, W1, b1, W2, bias2, blk):
    B, D = x.shape
    H = W1.shape[1]
    O = W2.shape[1]
    NT = bias2.shape[0]
    TD = O // NT
    grid = (B // blk,)
    return pl.pallas_call(
        _mlp_body,
        grid=grid,
        in_specs=[
            pl.BlockSpec((blk, D), lambda i: (i, 0)),
            pl.BlockSpec((D, H), lambda i: (0, 0)),
            pl.BlockSpec((1, H), lambda i: (0, 0)),
            pl.BlockSpec((H, O), lambda i: (0, 0)),
            pl.BlockSpec((NT, TD), lambda i: (0, 0)),
        ],
        out_specs=pl.BlockSpec(memory_space=pl.ANY),
        out_shape=jax.ShapeDtypeStruct((B, NT, TD), jnp.float32),
        scratch_shapes=[
            pltpu.VMEM((2, blk, TD), jnp.float32),
            pltpu.SemaphoreType.DMA((2,)),
        ],
    )(x, W1, b1, W2, bias2)


def kernel(labels, table, W1, b1, W2, b2, pos):
    B = labels.shape[0]
    V, D = table.shape
    NT, _ = pos.shape
    idx = labels.astype(jnp.int32)
    x = _make_sc_gather(V, D, B)(table, idx)
    bias2 = b2.reshape(NT, D) + pos
    return _tc_mlp(x, W1.astype(jnp.bfloat16), b1[None, :],
                   W2.astype(jnp.bfloat16), bias2, blk=1024)


# speedup vs baseline: 1.8889x; 1.8889x over previous
import functools

import jax
import jax.numpy as jnp
from jax import lax
from jax.experimental import pallas as pl
from jax.experimental.pallas import tpu as pltpu
from jax.experimental.pallas import tpu_sc as plsc


def _make_sc_gather(V, D, B):
    info = plsc.get_sparse_core_info()
    NC, NS = info.num_cores, info.num_subcores
    NW = NC * NS
    assert B % NW == 0
    b_per_w = B // NW
    CH = 128 if b_per_w % 128 == 0 else b_per_w
    n_ch = b_per_w // CH
    mesh = plsc.VectorSubcoreMesh(core_axis_name="c", subcore_axis_name="s")

    @functools.partial(
        pl.kernel,
        mesh=mesh,
        out_type=jax.ShapeDtypeStruct((B, D), jnp.float32),
        scratch_types=[
            pltpu.VMEM((b_per_w,), jnp.int32),
            pltpu.VMEM((b_per_w, D), jnp.float32),
            pltpu.SemaphoreType.DMA,
        ],
    )
    def sc_gather(table_hbm, idx_hbm, out_hbm, idx_v, rows_v, sem):
        wid = lax.axis_index("s") * NC + lax.axis_index("c")
        base = wid * b_per_w
        pltpu.sync_copy(idx_hbm.at[pl.ds(base, b_per_w)], idx_v)
        copies = []
        for j in range(n_ch):
            copies.append(pltpu.async_copy(
                table_hbm.at[idx_v.at[pl.ds(j * CH, CH)]],
                rows_v.at[pl.ds(j * CH, CH)],
                sem,
            ))
        for c in copies:
            c.wait()
        pltpu.sync_copy(rows_v, out_hbm.at[pl.ds(base, b_per_w)])

    return sc_gather


def _mlp_body(x_ref, w1_ref, b1_ref, w2_ref, b2_ref, o_ref):
    nt, td = b2_ref.shape
    x = x_ref[...].astype(jnp.bfloat16)
    h = jnp.dot(x, w1_ref[...], preferred_element_type=jnp.float32) + b1_ref[...]
    h = (h * jax.nn.sigmoid(h)).astype(jnp.bfloat16)
    o = jnp.dot(h, w2_ref[...], preferred_element_type=jnp.float32)
    o_ref[...] = o.reshape(o.shape[0], nt, td) + b2_ref[...]


def _tc_mlp(x, W1, b1, W2, bias2, blk):
    B, D = x.shape
    H = W1.shape[1]
    O = W2.shape[1]
    NT = bias2.shape[0]
    TD = O // NT
    grid = (B // blk,)
    return pl.pallas_call(
        _mlp_body,
        grid=grid,
        in_specs=[
            pl.BlockSpec((blk, D), lambda i: (i, 0)),
            pl.BlockSpec((D, H), lambda i: (0, 0)),
            pl.BlockSpec((1, H), lambda i: (0, 0)),
            pl.BlockSpec((H, O), lambda i: (0, 0)),
            pl.BlockSpec((NT, TD), lambda i: (0, 0)),
        ],
        out_specs=pl.BlockSpec((blk, NT, TD), lambda i: (i, 0, 0)),
        out_shape=jax.ShapeDtypeStruct((B, NT, TD), jnp.float32),
    )(x, W1, b1, W2, bias2)


def kernel(labels, table, W1, b1, W2, b2, pos):
    B = labels.shape[0]
    V, D = table.shape
    NT, _ = pos.shape
    idx = labels.astype(jnp.int32)
    x = _make_sc_gather(V, D, B)(table, idx)
    bias2 = b2.reshape(NT, D) + pos
    return _tc_mlp(x, W1.astype(jnp.bfloat16), b1[None, :],
                   W2.astype(jnp.bfloat16), bias2, blk=1024)


# speedup vs baseline: 1.9960x; 1.0567x over previous
import functools

import jax
import jax.numpy as jnp
from jax import lax
from jax.experimental import pallas as pl
from jax.experimental.pallas import tpu as pltpu
from jax.experimental.pallas import tpu_sc as plsc


def _make_sc_gather(V, D, B):
    info = plsc.get_sparse_core_info()
    NC, NS = info.num_cores, info.num_subcores
    NW = NC * NS
    assert B % NW == 0
    b_per_w = B // NW
    CH = 128 if b_per_w % 128 == 0 else b_per_w
    n_ch = b_per_w // CH
    mesh = plsc.VectorSubcoreMesh(core_axis_name="c", subcore_axis_name="s")

    @functools.partial(
        pl.kernel,
        mesh=mesh,
        out_type=jax.ShapeDtypeStruct((B, D), jnp.float32),
        scratch_types=[
            pltpu.VMEM((b_per_w,), jnp.int32),
            pltpu.VMEM((b_per_w, D), jnp.float32),
            pltpu.SemaphoreType.DMA,
        ],
    )
    def sc_gather(table_hbm, idx_hbm, out_hbm, idx_v, rows_v, sem):
        wid = lax.axis_index("s") * NC + lax.axis_index("c")
        base = wid * b_per_w
        pltpu.sync_copy(idx_hbm.at[pl.ds(base, b_per_w)], idx_v)
        copies = []
        for j in range(n_ch):
            copies.append(pltpu.async_copy(
                table_hbm.at[idx_v.at[pl.ds(j * CH, CH)]],
                rows_v.at[pl.ds(j * CH, CH)],
                sem,
            ))
        for c in copies:
            c.wait()
        pltpu.sync_copy(rows_v, out_hbm.at[pl.ds(base, b_per_w)])

    return sc_gather


def _mlp_body(x_ref, w1_ref, b1_ref, w2_ref, b2_ref, o_ref):
    nt, td = b2_ref.shape
    x = x_ref[...].astype(jnp.bfloat16)
    h = jnp.dot(x, w1_ref[...], preferred_element_type=jnp.float32) + b1_ref[...]
    h = (h * jax.nn.sigmoid(h)).astype(jnp.bfloat16)
    o = jnp.dot(h, w2_ref[...], preferred_element_type=jnp.float32)
    o_ref[...] = o.reshape(o.shape[0], nt, td) + b2_ref[...]


def _tc_mlp(x, W1, b1, W2, bias2, blk):
    B, D = x.shape
    H = W1.shape[1]
    O = W2.shape[1]
    NT = bias2.shape[0]
    TD = O // NT
    grid = (B // blk,)
    return pl.pallas_call(
        _mlp_body,
        grid=grid,
        in_specs=[
            pl.BlockSpec((blk, D), lambda i: (i, 0)),
            pl.BlockSpec((D, H), lambda i: (0, 0)),
            pl.BlockSpec((1, H), lambda i: (0, 0)),
            pl.BlockSpec((H, O), lambda i: (0, 0)),
            pl.BlockSpec((NT, TD), lambda i: (0, 0)),
        ],
        out_specs=pl.BlockSpec((blk, NT, TD), lambda i: (i, 0, 0)),
        out_shape=jax.ShapeDtypeStruct((B, NT, TD), jnp.float32),
    )(x, W1, b1, W2, bias2)


def kernel(labels, table, W1, b1, W2, b2, pos):
    B = labels.shape[0]
    V, D = table.shape
    NT, _ = pos.shape
    idx = labels.astype(jnp.int32)
    x = _make_sc_gather(V, D, B)(table, idx)
    bias2 = b2.reshape(NT, D) + pos
    return _tc_mlp(x, W1.astype(jnp.bfloat16), b1[None, :],
                   W2.astype(jnp.bfloat16), bias2, blk=2048)
